# Initial kernel scaffold; baseline (speedup 1.0000x reference)
import jax
import jax.numpy as jnp
from jax.experimental import pallas as pl


def kernel(x, edge_index, W1, b1, W2, b2, W3, b3):
    raise NotImplementedError("write your pallas kernel here")


# speedup vs baseline: 14.2072x; 14.2072x over previous
import functools

import jax
import jax.numpy as jnp
from jax import lax
from jax.experimental import pallas as pl
from jax.experimental.pallas import tpu as pltpu
from jax.experimental.pallas import tpu_sc as plsc

N = 10000
E = 320000
D = 128
NC = 2
NS = 16
NW = NC * NS
CHUNK = 128
NCHUNK = E // CHUNK
NPAD = 10240
SLICE = NPAD // NS
ZROWS = 64
BM = 1024
GRID = NPAD // BM

_mesh = plsc.VectorSubcoreMesh(core_axis_name="c", subcore_axis_name="s")


def _worker_bounds(wid):
    lo = (wid * NCHUNK) // NW
    hi = ((wid + 1) * NCHUNK) // NW
    return lo, hi


@functools.partial(
    pl.kernel,
    out_type=(
        jax.ShapeDtypeStruct((NPAD,), jnp.float32),
        jax.ShapeDtypeStruct((NPAD,), jnp.float32),
    ),
    mesh=_mesh,
    scratch_types=[
        pltpu.VMEM((CHUNK,), jnp.int32),
        pltpu.VMEM((CHUNK,), jnp.float32),
        pltpu.VMEM((SLICE,), jnp.float32),
        pltpu.VMEM_SHARED((NPAD,), jnp.float32),
    ],
)
def _deg_sc(dst_hbm, deg0_hbm, deg1_hbm, idx_v, ones_v, zero_v, acc_s):
    c = lax.axis_index("c")
    s = lax.axis_index("s")
    wid = s * NC + c

    def fill_ones(i, _):
        ones_v[pl.ds(i * 16, 16)] = jnp.ones((16,), jnp.float32)
        return 0

    def fill_zero(i, _):
        zero_v[pl.ds(i * 16, 16)] = jnp.zeros((16,), jnp.float32)
        return 0

    lax.fori_loop(0, CHUNK // 16, fill_ones, 0)
    lax.fori_loop(0, SLICE // 16, fill_zero, 0)
    pltpu.sync_copy(zero_v, acc_s.at[pl.ds(s * SLICE, SLICE)])
    plsc.subcore_barrier()

    lo, hi = _worker_bounds(wid)

    def step(ci, _):
        pltpu.sync_copy(dst_hbm.at[pl.ds(ci * CHUNK, CHUNK)], idx_v)
        pltpu.sync_copy(ones_v, acc_s.at[idx_v], add=True)
        return 0

    lax.fori_loop(lo, hi, step, 0)
    plsc.subcore_barrier()

    @pl.when(c == 0)
    def _():
        pltpu.sync_copy(acc_s.at[pl.ds(s * SLICE, SLICE)],
                        deg0_hbm.at[pl.ds(s * SLICE, SLICE)])

    @pl.when(c == 1)
    def _():
        pltpu.sync_copy(acc_s.at[pl.ds(s * SLICE, SLICE)],
                        deg1_hbm.at[pl.ds(s * SLICE, SLICE)])


@functools.partial(
    pl.kernel,
    out_type=(
        jax.ShapeDtypeStruct((NPAD, D), jnp.float32),
        jax.ShapeDtypeStruct((NPAD, D), jnp.float32),
    ),
    mesh=_mesh,
    scratch_types=[
        pltpu.VMEM((CHUNK,), jnp.int32),
        pltpu.VMEM((CHUNK,), jnp.int32),
        pltpu.VMEM((CHUNK, D), jnp.float32),
        pltpu.VMEM((ZROWS, D), jnp.float32),
        pltpu.VMEM_SHARED((NPAD, D), jnp.float32),
        pltpu.SemaphoreType.DMA,
    ],
)
def _agg_sc(hp_hbm, src_hbm, dst_hbm, acc0_hbm, acc1_hbm,
            src_v, dst_v, rows_v, zero_v, acc_s, sem):
    c = lax.axis_index("c")
    s = lax.axis_index("s")
    wid = s * NC + c

    def zrow(i, _):
        def zcol(j, _):
            zero_v[i, pl.ds(j * 16, 16)] = jnp.zeros((16,), jnp.float32)
            return 0
        lax.fori_loop(0, D // 16, zcol, 0)
        return 0

    lax.fori_loop(0, ZROWS, zrow, 0)
    for k in range(SLICE // ZROWS):
        pltpu.sync_copy(zero_v, acc_s.at[pl.ds(s * SLICE + k * ZROWS, ZROWS)])
    plsc.subcore_barrier()

    lo, hi = _worker_bounds(wid)

    def step(ci, _):
        base = ci * CHUNK
        pltpu.sync_copy(src_hbm.at[pl.ds(base, CHUNK)], src_v)
        pltpu.sync_copy(dst_hbm.at[pl.ds(base, CHUNK)], dst_v)
        pltpu.async_copy(hp_hbm.at[src_v], rows_v, sem).wait()
        pltpu.sync_copy(rows_v, acc_s.at[dst_v], add=True)
        return 0

    lax.fori_loop(lo, hi, step, 0)
    plsc.subcore_barrier()

    @pl.when(c == 0)
    def _():
        pltpu.sync_copy(acc_s.at[pl.ds(s * SLICE, SLICE)],
                        acc0_hbm.at[pl.ds(s * SLICE, SLICE)])

    @pl.when(c == 1)
    def _():
        pltpu.sync_copy(acc_s.at[pl.ds(s * SLICE, SLICE)],
                        acc1_hbm.at[pl.ds(s * SLICE, SLICE)])


def _row_block(i):
    return (i, 0)


def _mm_first_body(x_ref, w_ref, d0_ref, d1_ref, o_ref):
    dis = lax.rsqrt(d0_ref[...] + d1_ref[...] + 1.0)
    h = jnp.dot(x_ref[...], w_ref[...], preferred_element_type=jnp.float32)
    o_ref[...] = h * dis


def _mm_mid_body(a0_ref, a1_ref, hp_ref, d0_ref, d1_ref, b_ref, w_ref, o_ref):
    dis = lax.rsqrt(d0_ref[...] + d1_ref[...] + 1.0)
    xn = dis * (a0_ref[...] + a1_ref[...] + hp_ref[...]) + b_ref[...]
    xn = jnp.maximum(xn, 0.0)
    o_ref[...] = jnp.dot(xn, w_ref[...], preferred_element_type=jnp.float32) * dis


def _epi_body(a0_ref, a1_ref, hp_ref, d0_ref, d1_ref, b_ref, o_ref):
    dis = lax.rsqrt(d0_ref[...] + d1_ref[...] + 1.0)
    o_ref[...] = dis * (a0_ref[...] + a1_ref[...] + hp_ref[...]) + b_ref[...]


_vec_spec = pl.BlockSpec((BM, 1), _row_block)
_mat_spec = pl.BlockSpec((BM, D), _row_block)
_w_spec = pl.BlockSpec((D, D), lambda i: (0, 0))
_b_spec = pl.BlockSpec((1, D), lambda i: (0, 0))
_out_struct = jax.ShapeDtypeStruct((N, D), jnp.float32)


def _mm_first(x, w, d0, d1):
    return pl.pallas_call(
        _mm_first_body,
        grid=(GRID,),
        in_specs=[_mat_spec, _w_spec, _vec_spec, _vec_spec],
        out_specs=_mat_spec,
        out_shape=_out_struct,
    )(x, w, d0, d1)


def _mm_mid(a0, a1, hp, d0, d1, b, w):
    return pl.pallas_call(
        _mm_mid_body,
        grid=(GRID,),
        in_specs=[_mat_spec, _mat_spec, _mat_spec, _vec_spec, _vec_spec,
                  _b_spec, _w_spec],
        out_specs=_mat_spec,
        out_shape=_out_struct,
    )(a0, a1, hp, d0, d1, b, w)


def _epi(a0, a1, hp, d0, d1, b):
    return pl.pallas_call(
        _epi_body,
        grid=(GRID,),
        in_specs=[_mat_spec, _mat_spec, _mat_spec, _vec_spec, _vec_spec,
                  _b_spec],
        out_specs=_mat_spec,
        out_shape=_out_struct,
    )(a0, a1, hp, d0, d1, b)


def kernel(x, edge_index, W1, b1, W2, b2, W3, b3):
    src = edge_index[0].astype(jnp.int32)
    dst = edge_index[1].astype(jnp.int32)

    deg0, deg1 = _deg_sc(dst)
    d0 = deg0.reshape(NPAD, 1)
    d1 = deg1.reshape(NPAD, 1)
    b1r = b1.reshape(1, D)
    b2r = b2.reshape(1, D)
    b3r = b3.reshape(1, D)

    h1p = _mm_first(x, W1, d0, d1)
    a0, a1 = _agg_sc(h1p, src, dst)
    h2p = _mm_mid(a0, a1, h1p, d0, d1, b1r, W2)
    a0, a1 = _agg_sc(h2p, src, dst)
    h3p = _mm_mid(a0, a1, h2p, d0, d1, b2r, W3)
    a0, a1 = _agg_sc(h3p, src, dst)
    return _epi(a0, a1, h3p, d0, d1, b3r)


# speedup vs baseline: 28.5701x; 2.0110x over previous
import functools

import jax
import jax.numpy as jnp
from jax import lax
from jax.experimental import pallas as pl
from jax.experimental.pallas import tpu as pltpu
from jax.experimental.pallas import tpu_sc as plsc

N = 10000
E = 320000
D = 128
NC = 2
NS = 16
NW = NC * NS
CHUNK = 128
NCHPAD = 2560
CPW = NCHPAD // NW
EPAD = NCHPAD * CHUNK
NPAD = 10240
SLICE = NPAD // NS
ZROWS = 64
BM = 1024
GRID = NPAD // BM

_mesh = plsc.VectorSubcoreMesh(core_axis_name="c", subcore_axis_name="s")


@functools.partial(
    pl.kernel,
    out_type=(
        jax.ShapeDtypeStruct((NPAD,), jnp.float32),
        jax.ShapeDtypeStruct((NPAD,), jnp.float32),
    ),
    mesh=_mesh,
    scratch_types=[
        pltpu.VMEM((CPW, CHUNK), jnp.int32),
        pltpu.VMEM((CHUNK,), jnp.float32),
        pltpu.VMEM((SLICE,), jnp.float32),
        pltpu.VMEM_SHARED((NPAD,), jnp.float32),
    ],
)
def _deg_sc(dstp_hbm, deg0_hbm, deg1_hbm, idx_v, ones_v, zero_v, acc_s):
    c = lax.axis_index("c")
    s = lax.axis_index("s")
    wid = s * NC + c

    def fill_ones(i, _):
        ones_v[pl.ds(i * 16, 16)] = jnp.ones((16,), jnp.float32)
        return 0

    def fill_zero(i, _):
        zero_v[pl.ds(i * 16, 16)] = jnp.zeros((16,), jnp.float32)
        return 0

    lax.fori_loop(0, CHUNK // 16, fill_ones, 0)
    lax.fori_loop(0, SLICE // 16, fill_zero, 0)
    pltpu.sync_copy(zero_v, acc_s.at[pl.ds(s * SLICE, SLICE)])
    pltpu.sync_copy(dstp_hbm.at[pl.ds(wid * CPW, CPW)], idx_v)
    plsc.subcore_barrier()

    def step(ci, _):
        pltpu.sync_copy(ones_v, acc_s.at[idx_v.at[ci]], add=True)
        return 0

    lax.fori_loop(0, CPW, step, 0)
    plsc.subcore_barrier()

    @pl.when(c == 0)
    def _():
        pltpu.sync_copy(acc_s.at[pl.ds(s * SLICE, SLICE)],
                        deg0_hbm.at[pl.ds(s * SLICE, SLICE)])

    @pl.when(c == 1)
    def _():
        pltpu.sync_copy(acc_s.at[pl.ds(s * SLICE, SLICE)],
                        deg1_hbm.at[pl.ds(s * SLICE, SLICE)])


@functools.partial(
    pl.kernel,
    out_type=(
        jax.ShapeDtypeStruct((NPAD, D), jnp.float32),
        jax.ShapeDtypeStruct((NPAD, D), jnp.float32),
    ),
    mesh=_mesh,
    scratch_types=[
        pltpu.VMEM((CPW // 2, CHUNK), jnp.int32),
        pltpu.VMEM((CPW // 2, CHUNK), jnp.int32),
        pltpu.VMEM((2, CHUNK, D), jnp.float32),
        pltpu.VMEM_SHARED((NPAD, D), jnp.float32),
        pltpu.SemaphoreType.DMA,
        pltpu.SemaphoreType.DMA,
    ],
)
def _agg_sc(hp_hbm, srcp_hbm, dstp_hbm, acc0_hbm, acc1_hbm,
            src_v, dst_v, rows_v, acc_s, gsem0, gsem1):
    c = lax.axis_index("c")
    s = lax.axis_index("s")
    wid = s * NC + c
    half = CPW // 2

    def zrow(i, _):
        def zcol(j, _):
            rows_v[0, i, pl.ds(j * 16, 16)] = jnp.zeros((16,), jnp.float32)
            return 0
        lax.fori_loop(0, D // 16, zcol, 0)
        return 0

    lax.fori_loop(0, CHUNK, zrow, 0)
    for k in range(SLICE // CHUNK):
        pltpu.sync_copy(rows_v.at[0],
                        acc_s.at[pl.ds(s * SLICE + k * CHUNK, CHUNK)])
    plsc.subcore_barrier()

    gsems = (gsem0, gsem1)

    def gather_start(ci, b):
        pltpu.async_copy(hp_hbm.at[src_v.at[ci]], rows_v.at[b], gsems[b])

    def gather_wait(ci, b):
        pltpu.make_async_copy(hp_hbm.at[src_v.at[ci]], rows_v.at[b],
                              gsems[b]).wait()

    for h in range(2):
        pltpu.sync_copy(srcp_hbm.at[pl.ds(wid * CPW + h * half, half)], src_v)
        pltpu.sync_copy(dstp_hbm.at[pl.ds(wid * CPW + h * half, half)], dst_v)
        for b in range(2):
            gather_start(b, b)

        def outer(g, _):
            for b in range(2):
                ci = 2 * g + b
                gather_wait(ci, b)
                pltpu.sync_copy(rows_v.at[b], acc_s.at[dst_v.at[ci]],
                                add=True)
                gather_start(ci + 2, b)
            return 0

        lax.fori_loop(0, half // 2 - 1, outer, 0)
        for b in range(2):
            ci = half - 2 + b
            gather_wait(ci, b)
            pltpu.sync_copy(rows_v.at[b], acc_s.at[dst_v.at[ci]], add=True)
    plsc.subcore_barrier()

    @pl.when(c == 0)
    def _():
        pltpu.sync_copy(acc_s.at[pl.ds(s * SLICE, SLICE)],
                        acc0_hbm.at[pl.ds(s * SLICE, SLICE)])

    @pl.when(c == 1)
    def _():
        pltpu.sync_copy(acc_s.at[pl.ds(s * SLICE, SLICE)],
                        acc1_hbm.at[pl.ds(s * SLICE, SLICE)])


def _row_block(i):
    return (i, 0)


def _mm_first_body(x_ref, w_ref, d0_ref, d1_ref, o_ref):
    dis = lax.rsqrt(d0_ref[...] + d1_ref[...] + 1.0)
    h = jnp.dot(x_ref[...], w_ref[...], preferred_element_type=jnp.float32)
    o_ref[...] = h * dis


def _mm_mid_body(a0_ref, a1_ref, hp_ref, d0_ref, d1_ref, b_ref, w_ref, o_ref):
    dis = lax.rsqrt(d0_ref[...] + d1_ref[...] + 1.0)
    xn = dis * (a0_ref[...] + a1_ref[...] + hp_ref[...]) + b_ref[...]
    xn = jnp.maximum(xn, 0.0)
    o_ref[...] = jnp.dot(xn, w_ref[...], preferred_element_type=jnp.float32) * dis


def _epi_body(a0_ref, a1_ref, hp_ref, d0_ref, d1_ref, b_ref, o_ref):
    dis = lax.rsqrt(d0_ref[...] + d1_ref[...] + 1.0)
    o_ref[...] = dis * (a0_ref[...] + a1_ref[...] + hp_ref[...]) + b_ref[...]


_vec_spec = pl.BlockSpec((BM, 1), _row_block)
_mat_spec = pl.BlockSpec((BM, D), _row_block)
_w_spec = pl.BlockSpec((D, D), lambda i: (0, 0))
_b_spec = pl.BlockSpec((1, D), lambda i: (0, 0))
_out_struct = jax.ShapeDtypeStruct((N, D), jnp.float32)


def _mm_first(x, w, d0, d1):
    return pl.pallas_call(
        _mm_first_body,
        grid=(GRID,),
        in_specs=[_mat_spec, _w_spec, _vec_spec, _vec_spec],
        out_specs=_mat_spec,
        out_shape=_out_struct,
    )(x, w, d0, d1)


def _mm_mid(a0, a1, hp, d0, d1, b, w):
    return pl.pallas_call(
        _mm_mid_body,
        grid=(GRID,),
        in_specs=[_mat_spec, _mat_spec, _mat_spec, _vec_spec, _vec_spec,
                  _b_spec, _w_spec],
        out_specs=_mat_spec,
        out_shape=_out_struct,
    )(a0, a1, hp, d0, d1, b, w)


def _epi(a0, a1, hp, d0, d1, b):
    return pl.pallas_call(
        _epi_body,
        grid=(GRID,),
        in_specs=[_mat_spec, _mat_spec, _mat_spec, _vec_spec, _vec_spec,
                  _b_spec],
        out_specs=_mat_spec,
        out_shape=_out_struct,
    )(a0, a1, hp, d0, d1, b)


def kernel(x, edge_index, W1, b1, W2, b2, W3, b3):
    src = edge_index[0].astype(jnp.int32)
    dst = edge_index[1].astype(jnp.int32)

    npad_e = EPAD - E
    pad_ar = jnp.arange(npad_e, dtype=jnp.int32)
    pad_src = (pad_ar * 61) % N
    pad_dst = N + pad_ar % (NPAD - N)
    srcp = jnp.concatenate([src, pad_src]).reshape(NCHPAD, CHUNK)
    dstp = jnp.concatenate([dst, pad_dst]).reshape(NCHPAD, CHUNK)

    deg0, deg1 = _deg_sc(dstp)
    d0 = deg0.reshape(NPAD, 1)
    d1 = deg1.reshape(NPAD, 1)
    b1r = b1.reshape(1, D)
    b2r = b2.reshape(1, D)
    b3r = b3.reshape(1, D)

    h1p = _mm_first(x, W1, d0, d1)
    a0, a1 = _agg_sc(h1p, srcp, dstp)
    h2p = _mm_mid(a0, a1, h1p, d0, d1, b1r, W2)
    a0, a1 = _agg_sc(h2p, srcp, dstp)
    h3p = _mm_mid(a0, a1, h2p, d0, d1, b2r, W3)
    a0, a1 = _agg_sc(h3p, srcp, dstp)
    return _epi(a0, a1, h3p, d0, d1, b3r)
